# Initial kernel scaffold; baseline (speedup 1.0000x reference)
import jax
import jax.numpy as jnp
from jax.experimental import pallas as pl


def kernel(x, edge_index, params):
    raise NotImplementedError("write your pallas kernel here")


# speedup vs baseline: 6.3046x; 6.3046x over previous
import functools

import jax
import jax.numpy as jnp
from jax import lax
from jax.experimental import pallas as pl
from jax.experimental.pallas import tpu as pltpu
from jax.experimental.pallas import tpu_sc as plsc

N_NODES = 10000
N_EDGES = 320000
D = 128
MLP_H = 256
EPS = 1e-5

NC = 2
NS = 16
NW = NC * NS
E_PER_TILE = N_EDGES // NW
K = 80
CHUNKS = E_PER_TILE // K
ROWS_PER_TILE = N_NODES // NS
CNT_TILES = 5
CNT_PER_TILE = N_NODES // CNT_TILES


def _sc_agg_body(compute_cnt, *refs):
    if compute_cnt:
        (h_hbm, src_hbm, dst_hbm, zf_hbm, zc_hbm,
         agg_out, cnt_out,
         src_v, dst_v, rows_v, agg_sp, cnt_sp, cnt_v, sem) = refs
    else:
        (h_hbm, src_hbm, dst_hbm, zf_hbm,
         agg_out,
         src_v, dst_v, rows_v, agg_sp, sem) = refs

    c = lax.axis_index("c")
    s = lax.axis_index("s")
    wid = c * NS + s

    pltpu.sync_copy(zf_hbm, agg_sp.at[pl.ds(s * ROWS_PER_TILE, ROWS_PER_TILE)])
    if compute_cnt:
        @pl.when(s < CNT_TILES)
        def _():
            pltpu.sync_copy(zc_hbm, cnt_v)
            pltpu.sync_copy(cnt_v, cnt_sp.at[pl.ds(s * CNT_PER_TILE, CNT_PER_TILE)])
    plsc.subcore_barrier()

    pltpu.sync_copy(src_hbm.at[wid], src_v)
    pltpu.sync_copy(dst_hbm.at[wid], dst_v)

    if compute_cnt:
        ones = jnp.ones((16,), jnp.float32)
        for i in range(K // 16):
            rows_v[0, pl.ds(i * 16, 16)] = ones

    def step(j, carry):
        pltpu.async_copy(h_hbm.at[src_v.at[j]], rows_v, sem).wait()
        pltpu.sync_copy(rows_v, agg_sp.at[dst_v.at[j]], add=True)
        return carry

    def step_cnt(j, carry):
        pltpu.sync_copy(rows_v.at[0, pl.ds(0, K)], cnt_sp.at[dst_v.at[j]], add=True)
        return carry

    if compute_cnt:
        lax.fori_loop(0, CHUNKS, step_cnt, 0)
    lax.fori_loop(0, CHUNKS, step, 0)
    plsc.subcore_barrier()

    pltpu.sync_copy(agg_sp.at[pl.ds(s * ROWS_PER_TILE, ROWS_PER_TILE)],
                    agg_out.at[c, s])
    if compute_cnt:
        @pl.when(s < CNT_TILES)
        def _():
            pltpu.sync_copy(cnt_sp.at[pl.ds(s * CNT_PER_TILE, CNT_PER_TILE)], cnt_v)
            pltpu.sync_copy(cnt_v, cnt_out.at[c, s])


def _make_sc_agg(compute_cnt):
    out_type = [jax.ShapeDtypeStruct((NC, NS, ROWS_PER_TILE, D), jnp.float32)]
    scratch = [
        pltpu.VMEM((CHUNKS, K), jnp.int32),
        pltpu.VMEM((CHUNKS, K), jnp.int32),
        pltpu.VMEM((K, D), jnp.float32),
        pltpu.VMEM_SHARED((N_NODES, D), jnp.float32),
    ]
    if compute_cnt:
        out_type.append(jax.ShapeDtypeStruct((NC, CNT_TILES, CNT_PER_TILE), jnp.float32))
        scratch.append(pltpu.VMEM_SHARED((N_NODES,), jnp.float32))
        scratch.append(pltpu.VMEM((CNT_PER_TILE,), jnp.float32))
    scratch.append(pltpu.SemaphoreType.DMA)
    mesh = plsc.VectorSubcoreMesh(core_axis_name="c", subcore_axis_name="s",
                                  num_cores=NC, num_subcores=NS)
    return pl.kernel(functools.partial(_sc_agg_body, compute_cnt),
                     out_type=tuple(out_type), mesh=mesh,
                     scratch_types=scratch)


_sc_agg_cnt = _make_sc_agg(True)
_sc_agg = _make_sc_agg(False)


def _tc_layer_body(agg_ref, cnt_ref, h_ref, wl_ref, bl_ref, wr_ref,
                   gw_ref, gb_ref, ga_ref, out_ref):
    agg = agg_ref[0] + agg_ref[1]
    cnt = jnp.maximum(cnt_ref[0] + cnt_ref[1], 1.0)
    mean = agg / cnt
    pre = (jnp.dot(mean, wl_ref[...], preferred_element_type=jnp.float32)
           + bl_ref[...]
           + jnp.dot(h_ref[...], wr_ref[...], preferred_element_type=jnp.float32))
    mu = jnp.mean(pre, axis=0, keepdims=True)
    cen = pre - ga_ref[...] * mu
    var = jnp.mean(cen * cen, axis=0, keepdims=True)
    out = gw_ref[...] * cen * lax.rsqrt(var + EPS) + gb_ref[...]
    out_ref[...] = jnp.maximum(out, 0.0)


_tc_layer = pl.pallas_call(
    _tc_layer_body,
    out_shape=jax.ShapeDtypeStruct((N_NODES, D), jnp.float32),
)


def _tc_heads_body(h_ref, w1a, b1a, w2a, b2a, w1b, b1b, w2b, b2b,
                   w1c, b1c, w2c, b2c, oa, ob, oc):
    h = h_ref[...]

    def head(w1, b1, w2, b2, o):
        z = jnp.maximum(jnp.dot(h, w1[...], preferred_element_type=jnp.float32)
                        + b1[...], 0.0)
        o[...] = jnp.dot(z, w2[...], preferred_element_type=jnp.float32) + b2[...]

    head(w1a, b1a, w2a, b2a, oa)
    head(w1b, b1b, w2b, b2b, ob)
    head(w1c, b1c, w2c, b2c, oc)


def _make_heads(n_hh, douts):
    return pl.pallas_call(
        _tc_heads_body,
        out_shape=tuple(jax.ShapeDtypeStruct((n_hh, d), jnp.float32)
                        for d in douts),
    )


def kernel(x, edge_index, params):
    src = edge_index[0].astype(jnp.int32).reshape(NW, CHUNKS, K)
    dst = edge_index[1].astype(jnp.int32).reshape(NW, CHUNKS, K)
    zf = jnp.zeros((ROWS_PER_TILE, D), jnp.float32)
    zc = jnp.zeros((CNT_PER_TILE,), jnp.float32)

    h = x
    cnt = None
    for i in (1, 2, 3):
        if i == 1:
            agg_p, cnt_p = _sc_agg_cnt(h, src, dst, zf, zc)
            cnt = cnt_p.reshape(NC, N_NODES, 1)
        else:
            (agg_p,) = _sc_agg(h, src, dst, zf)
        agg_p = agg_p.reshape(NC, N_NODES, D)
        h = _tc_layer(agg_p, cnt,
                      h,
                      params['W%dl' % i], params['b%dl' % i].reshape(1, D),
                      params['W%dr' % i],
                      params['g%dw' % i].reshape(1, D),
                      params['g%db' % i].reshape(1, D),
                      params['g%da' % i].reshape(1, D))

    hh_in = h[:8192]
    heads = _make_heads(8192, (12, 18, 9))
    hh_out, eth_out, rel_out = heads(
        hh_in,
        params['hhW1'], params['hhb1'].reshape(1, MLP_H),
        params['hhW2'], params['hhb2'].reshape(1, 12),
        params['etW1'], params['etb1'].reshape(1, MLP_H),
        params['etW2'], params['etb2'].reshape(1, 18),
        params['rlW1'], params['rlb1'].reshape(1, MLP_H),
        params['rlW2'], params['rlb2'].reshape(1, 9),
    )
    return (hh_out, eth_out, rel_out)
